# Initial kernel scaffold; baseline (speedup 1.0000x reference)
import jax
import jax.numpy as jnp
from jax.experimental import pallas as pl


def kernel(x, edge_index, edge_attr, initial_importance, batch, imp_proj_W, imp_proj_b, init_W, init_b, conv_lin_W, conv_lin_b, edge_lin_W, edge_lin_b, gate_W, gate_b, prop_W, prop_b, gn_weight, gn_bias, gn_mean_scale, final_W, final_b):
    raise NotImplementedError("write your pallas kernel here")


# speedup vs baseline: 2.0078x; 2.0078x over previous
import functools

import jax
import jax.numpy as jnp
from jax import lax
from jax.experimental import pallas as pl
from jax.experimental.pallas import tpu as pltpu
from jax.experimental.pallas import tpu_sc as plsc

_N = 10000
_E = 160000
_D = 256
_H = 256
_ED = 16
_L = 3
_OUT = 256
_B = 8
_DECAY = 0.9
_THRESH = 0.1
_HH = 128
_BIG = 3.0e38

_BN = 2000
_NB = _N // _BN
_BE = 4000
_NEB = _E // _BE

_NS = 16
_EPT = _E // _NS
_CH = 80
_NCH = _EPT // _CH
_RPT = 632
_NPAD = _NS * _RPT
_RLAST = _N - 15 * _RPT


def _lazy_norm(raw, mn_row, mx_row, oh):
    mnb = jnp.sum(oh * mn_row, axis=1, keepdims=True)
    mxb = jnp.sum(oh * mx_row, axis=1, keepdims=True)
    return (raw - mnb) / (mxb - mnb + 1e-8)


def _sigmoid(z):
    return 1.0 / (1.0 + jnp.exp(-z))


def _segdot(oh, x):
    return lax.dot_general(oh, x, (((0,), (0,)), ((), ())),
                           preferred_element_type=jnp.float32)


def _sc_agg(h_split, e_split, src, dst):
    mesh = plsc.VectorSubcoreMesh(core_axis_name="c", subcore_axis_name="s")

    @functools.partial(
        pl.kernel,
        out_type=jax.ShapeDtypeStruct((2 * _N, _HH), jnp.float32),
        mesh=mesh,
        scratch_types=[
            pltpu.VMEM((_CH,), jnp.int32),
            pltpu.VMEM((_CH,), jnp.int32),
            pltpu.VMEM((_CH, _HH), jnp.float32),
            pltpu.VMEM((_CH, _HH), jnp.float32),
            pltpu.VMEM_SHARED((_NPAD, _HH), jnp.float32),
            pltpu.SemaphoreType.DMA,
            pltpu.SemaphoreType.DMA,
        ],
    )
    def k(h_hbm, e_hbm, src_hbm, dst_hbm, agg_hbm,
          src_v, dst_v, rows_v, e_v, agg_sh, sem1, sem2):
        c = lax.axis_index("c")
        s = lax.axis_index("s")

        def zf(i, _):
            def zg(j, _):
                rows_v[i, pl.ds(j * 16, 16)] = jnp.zeros((16,), jnp.float32)
                return 0
            return lax.fori_loop(0, _HH // 16, zg, 0)
        lax.fori_loop(0, _CH, zf, 0)

        base_r = s * _RPT
        def zc(i, _):
            pltpu.sync_copy(rows_v, agg_sh.at[pl.ds(base_r + i * _CH, _CH)])
            return 0
        lax.fori_loop(0, _RPT // _CH, zc, 0)
        rem = _RPT - (_RPT // _CH) * _CH
        if rem:
            pltpu.sync_copy(rows_v.at[pl.ds(0, rem)],
                            agg_sh.at[pl.ds(base_r + _RPT - rem, rem)])
        plsc.subcore_barrier()
        del base_r

        cN = c * _N
        cE = c * _E
        ebase = s * _EPT

        def chunk(kk, _):
            b = ebase + kk * _CH
            pltpu.sync_copy(src_hbm.at[pl.ds(b, _CH)], src_v)
            pltpu.sync_copy(dst_hbm.at[pl.ds(b, _CH)], dst_v)
            for j in range(_CH // 16):
                src_v[pl.ds(j * 16, 16)] = src_v[pl.ds(j * 16, 16)] + cN
            cp1 = pltpu.async_copy(h_hbm.at[src_v], rows_v, sem1)
            cp2 = pltpu.async_copy(e_hbm.at[pl.ds(cE + b, _CH)], e_v, sem2)
            cp2.wait()
            cp1.wait()

            def fc(i, _):
                def fi(j, _):
                    o = j * 16
                    rows_v[i, pl.ds(o, 16)] = jnp.maximum(
                        rows_v[i, pl.ds(o, 16)] + e_v[i, pl.ds(o, 16)], 0.0)
                    return 0
                return lax.fori_loop(0, _HH // 16, fi, 0)
            lax.fori_loop(0, _CH, fc, 0)

            pltpu.sync_copy(rows_v, agg_sh.at[dst_v], add=True)
            return 0
        lax.fori_loop(0, _NCH, chunk, 0)

        plsc.subcore_barrier()

        @pl.when(s < _NS - 1)
        def _():
            pltpu.sync_copy(agg_sh.at[pl.ds(s * _RPT, _RPT)],
                            agg_hbm.at[pl.ds(cN + s * _RPT, _RPT)])

        @pl.when(s == _NS - 1)
        def _():
            pltpu.sync_copy(agg_sh.at[pl.ds(15 * _RPT, _RLAST)],
                            agg_hbm.at[pl.ds(cN + 15 * _RPT, _RLAST)])

    return k(h_split, e_split, src, dst)


def _edge_proj(edge_attr, We, be):
    def body(ea_ref, w_ref, b_ref, out_ref):
        out_ref[...] = (jnp.dot(ea_ref[...], w_ref[0],
                                preferred_element_type=jnp.float32)
                        + b_ref[0])[None]
    return pl.pallas_call(
        body,
        grid=(2 * _L, _NEB),
        in_specs=[
            pl.BlockSpec((_BE, _ED), lambda p, i: (i, 0)),
            pl.BlockSpec((1, _ED, _HH), lambda p, i: (p, 0, 0)),
            pl.BlockSpec((1, 1, _HH), lambda p, i: (p, 0, 0)),
        ],
        out_specs=pl.BlockSpec((1, _BE, _HH), lambda p, i: (p, i, 0)),
        out_shape=jax.ShapeDtypeStruct((2 * _L, _E, _HH), jnp.float32),
    )(edge_attr, We, be)


def _seg_stats0(imp0, oh):
    def body(imp_ref, oh_ref, mn_ref, mx_ref, cnt_ref):
        i = pl.program_id(0)
        @pl.when(i == 0)
        def _():
            mn_ref[...] = jnp.full((1, _B), _BIG, jnp.float32)
            mx_ref[...] = jnp.full((1, _B), -_BIG, jnp.float32)
            cnt_ref[...] = jnp.zeros((1, _B), jnp.float32)
        imp = imp_ref[...]
        oh = oh_ref[...]
        mn_ref[...] = jnp.minimum(
            mn_ref[...],
            jnp.min(jnp.where(oh > 0, imp, _BIG), axis=0, keepdims=True))
        mx_ref[...] = jnp.maximum(
            mx_ref[...],
            jnp.max(jnp.where(oh > 0, imp, -_BIG), axis=0, keepdims=True))
        cnt_ref[...] += jnp.sum(oh, axis=0, keepdims=True)
    return pl.pallas_call(
        body,
        grid=(_NB,),
        in_specs=[
            pl.BlockSpec((_BN, 1), lambda i: (i, 0)),
            pl.BlockSpec((_BN, _B), lambda i: (i, 0)),
        ],
        out_specs=[
            pl.BlockSpec((1, _B), lambda i: (0, 0)),
            pl.BlockSpec((1, _B), lambda i: (0, 0)),
            pl.BlockSpec((1, _B), lambda i: (0, 0)),
        ],
        out_shape=[
            jax.ShapeDtypeStruct((1, _B), jnp.float32),
            jax.ShapeDtypeStruct((1, _B), jnp.float32),
            jax.ShapeDtypeStruct((1, _B), jnp.float32),
        ],
    )(imp0, oh)


def _init_h(x, imp0, mn0, mx0, oh, projW, projb, Wx, Wi, initb):
    def body(x_ref, imp_ref, mn_ref, mx_ref, oh_ref, pw_ref, pb_ref,
             wx_ref, wi_ref, b_ref, out_ref):
        imp = _lazy_norm(imp_ref[...], mn_ref[...], mx_ref[...], oh_ref[...])
        emb = imp * pw_ref[...] + pb_ref[...]
        out_ref[...] = (jnp.dot(x_ref[...], wx_ref[...],
                                preferred_element_type=jnp.float32)
                        + jnp.dot(emb, wi_ref[...],
                                  preferred_element_type=jnp.float32)
                        + b_ref[...])
    return pl.pallas_call(
        body,
        grid=(2, _NB),
        in_specs=[
            pl.BlockSpec((_BN, _D), lambda c, i: (i, 0)),
            pl.BlockSpec((_BN, 1), lambda c, i: (i, 0)),
            pl.BlockSpec((1, _B), lambda c, i: (0, 0)),
            pl.BlockSpec((1, _B), lambda c, i: (0, 0)),
            pl.BlockSpec((_BN, _B), lambda c, i: (i, 0)),
            pl.BlockSpec((1, _H), lambda c, i: (0, 0)),
            pl.BlockSpec((1, _H), lambda c, i: (0, 0)),
            pl.BlockSpec((_D, _HH), lambda c, i: (0, c)),
            pl.BlockSpec((_H, _HH), lambda c, i: (0, c)),
            pl.BlockSpec((1, _HH), lambda c, i: (0, c)),
        ],
        out_specs=pl.BlockSpec((_BN, _HH), lambda c, i: (c * _NB + i, 0)),
        out_shape=jax.ShapeDtypeStruct((2 * _N, _HH), jnp.float32),
    )(x, imp0, mn0, mx0, oh, projW, projb, Wx, Wi, initb)


def _layer_k1(hs, aggs, imp_raw, mn, mx, oh,
              convW, convb, gWh, gWi, gb, propWr, propb):
    def body(hlo, hhi, alo, ahi, imp_ref, mn_ref, mx_ref, oh_ref,
             cw, cb, gwh, gwi, gbr, pwr, pbr, out_ref, prop_ref, seg_ref):
        i = pl.program_id(0)
        h = jnp.concatenate([hlo[...], hhi[...]], axis=1)
        agg = jnp.concatenate([alo[...], ahi[...]], axis=1)
        imp = _lazy_norm(imp_ref[...], mn_ref[...], mx_ref[...], oh_ref[...])
        conv = jnp.dot(h + agg, cw[...],
                       preferred_element_type=jnp.float32) + cb[...]
        gate = _sigmoid(jnp.dot(conv, gwh[...],
                                preferred_element_type=jnp.float32)
                        + imp * gwi[...] + gbr[...])
        o = gate * conv + (1.0 - gate) * h
        out_ref[...] = o
        prop_ref[...] = jnp.sum(o * pwr[...], axis=1, keepdims=True) + pbr[...]
        @pl.when(i == 0)
        def _():
            seg_ref[...] = jnp.zeros((_B, _H), jnp.float32)
        seg_ref[...] += _segdot(oh_ref[...], o)
    return pl.pallas_call(
        body,
        grid=(_NB,),
        in_specs=[
            pl.BlockSpec((_BN, _HH), lambda i: (i, 0)),
            pl.BlockSpec((_BN, _HH), lambda i: (_NB + i, 0)),
            pl.BlockSpec((_BN, _HH), lambda i: (i, 0)),
            pl.BlockSpec((_BN, _HH), lambda i: (_NB + i, 0)),
            pl.BlockSpec((_BN, 1), lambda i: (i, 0)),
            pl.BlockSpec((1, _B), lambda i: (0, 0)),
            pl.BlockSpec((1, _B), lambda i: (0, 0)),
            pl.BlockSpec((_BN, _B), lambda i: (i, 0)),
            pl.BlockSpec((_H, _H), lambda i: (0, 0)),
            pl.BlockSpec((1, _H), lambda i: (0, 0)),
            pl.BlockSpec((_H, _H), lambda i: (0, 0)),
            pl.BlockSpec((1, _H), lambda i: (0, 0)),
            pl.BlockSpec((1, _H), lambda i: (0, 0)),
            pl.BlockSpec((1, _H), lambda i: (0, 0)),
            pl.BlockSpec((1, 1), lambda i: (0, 0)),
        ],
        out_specs=[
            pl.BlockSpec((_BN, _H), lambda i: (i, 0)),
            pl.BlockSpec((_BN, 1), lambda i: (i, 0)),
            pl.BlockSpec((_B, _H), lambda i: (0, 0)),
        ],
        out_shape=[
            jax.ShapeDtypeStruct((_N, _H), jnp.float32),
            jax.ShapeDtypeStruct((_N, 1), jnp.float32),
            jax.ShapeDtypeStruct((_B, _H), jnp.float32),
        ],
    )(hs, hs, aggs, aggs, imp_raw, mn, mx, oh,
      convW, convb, gWh, gWi, gb, propWr, propb)


def _layer_k2(out, seg, cnt, oh, gn_ms, imp_raw, mn, mx, prop):
    def body(out_ref, seg_ref, cnt_ref, oh_ref, ms_ref,
             imp_ref, mn_ref, mx_ref, prop_ref,
             cent_ref, seg2_ref, mna_ref, mxa_ref, imp2_ref):
        i = pl.program_id(0)
        mean = seg_ref[...] / cnt_ref[...]
        meanb = jnp.dot(oh_ref[...], mean, preferred_element_type=jnp.float32)
        cent = out_ref[...] - meanb * ms_ref[...]
        cent_ref[...] = cent
        imp = _lazy_norm(imp_ref[...], mn_ref[...], mx_ref[...], oh_ref[...])
        imp2 = _DECAY * imp + (1.0 - _DECAY) * prop_ref[...]
        imp2_ref[...] = imp2
        oh = oh_ref[...]
        @pl.when(i == 0)
        def _():
            seg2_ref[...] = jnp.zeros((_B, _H), jnp.float32)
            mna_ref[...] = jnp.full((1, _B), _BIG, jnp.float32)
            mxa_ref[...] = jnp.full((1, _B), -_BIG, jnp.float32)
        seg2_ref[...] += _segdot(oh, cent * cent)
        mna_ref[...] = jnp.minimum(
            mna_ref[...],
            jnp.min(jnp.where(oh > 0, imp2, _BIG), axis=0, keepdims=True))
        mxa_ref[...] = jnp.maximum(
            mxa_ref[...],
            jnp.max(jnp.where(oh > 0, imp2, -_BIG), axis=0, keepdims=True))
    return pl.pallas_call(
        body,
        grid=(_NB,),
        in_specs=[
            pl.BlockSpec((_BN, _H), lambda i: (i, 0)),
            pl.BlockSpec((_B, _H), lambda i: (0, 0)),
            pl.BlockSpec((_B, 1), lambda i: (0, 0)),
            pl.BlockSpec((_BN, _B), lambda i: (i, 0)),
            pl.BlockSpec((1, _H), lambda i: (0, 0)),
            pl.BlockSpec((_BN, 1), lambda i: (i, 0)),
            pl.BlockSpec((1, _B), lambda i: (0, 0)),
            pl.BlockSpec((1, _B), lambda i: (0, 0)),
            pl.BlockSpec((_BN, 1), lambda i: (i, 0)),
        ],
        out_specs=[
            pl.BlockSpec((_BN, _H), lambda i: (i, 0)),
            pl.BlockSpec((_B, _H), lambda i: (0, 0)),
            pl.BlockSpec((1, _B), lambda i: (0, 0)),
            pl.BlockSpec((1, _B), lambda i: (0, 0)),
            pl.BlockSpec((_BN, 1), lambda i: (i, 0)),
        ],
        out_shape=[
            jax.ShapeDtypeStruct((_N, _H), jnp.float32),
            jax.ShapeDtypeStruct((_B, _H), jnp.float32),
            jax.ShapeDtypeStruct((1, _B), jnp.float32),
            jax.ShapeDtypeStruct((1, _B), jnp.float32),
            jax.ShapeDtypeStruct((_N, 1), jnp.float32),
        ],
    )(out, seg, cnt, oh, gn_ms, imp_raw, mn, mx, prop)


def _layer_k3(cent, seg2, cnt, oh, gn_w, gn_b, imp2, mna, mxa):
    def body(cent_ref, seg2_ref, cnt_ref, oh_ref, gw_ref, gb_ref,
             imp2_ref, mna_ref, mxa_ref,
             hs_ref, imp3_ref, mnb_ref, mxb_ref):
        c = pl.program_id(0)
        i = pl.program_id(1)
        var = seg2_ref[...] / cnt_ref[...]
        varb = jnp.dot(oh_ref[...], var, preferred_element_type=jnp.float32)
        normed = cent_ref[...] * lax.rsqrt(varb + 1e-5) * gw_ref[...] + gb_ref[...]
        hs_ref[...] = jnp.maximum(normed, 0.0)
        imp2n = _lazy_norm(imp2_ref[...], mna_ref[...], mxa_ref[...], oh_ref[...])
        imp3 = jnp.where(imp2n < _THRESH, 0.0, imp2n)
        imp3_ref[...] = imp3
        oh = oh_ref[...]
        @pl.when((c == 0) & (i == 0))
        def _():
            mnb_ref[...] = jnp.full((1, _B), _BIG, jnp.float32)
            mxb_ref[...] = jnp.full((1, _B), -_BIG, jnp.float32)
        mnb_ref[...] = jnp.minimum(
            mnb_ref[...],
            jnp.min(jnp.where(oh > 0, imp3, _BIG), axis=0, keepdims=True))
        mxb_ref[...] = jnp.maximum(
            mxb_ref[...],
            jnp.max(jnp.where(oh > 0, imp3, -_BIG), axis=0, keepdims=True))
    return pl.pallas_call(
        body,
        grid=(2, _NB),
        in_specs=[
            pl.BlockSpec((_BN, _HH), lambda c, i: (i, c)),
            pl.BlockSpec((_B, _HH), lambda c, i: (0, c)),
            pl.BlockSpec((_B, 1), lambda c, i: (0, 0)),
            pl.BlockSpec((_BN, _B), lambda c, i: (i, 0)),
            pl.BlockSpec((1, _HH), lambda c, i: (0, c)),
            pl.BlockSpec((1, _HH), lambda c, i: (0, c)),
            pl.BlockSpec((_BN, 1), lambda c, i: (i, 0)),
            pl.BlockSpec((1, _B), lambda c, i: (0, 0)),
            pl.BlockSpec((1, _B), lambda c, i: (0, 0)),
        ],
        out_specs=[
            pl.BlockSpec((_BN, _HH), lambda c, i: (c * _NB + i, 0)),
            pl.BlockSpec((_BN, 1), lambda c, i: (i, 0)),
            pl.BlockSpec((1, _B), lambda c, i: (0, 0)),
            pl.BlockSpec((1, _B), lambda c, i: (0, 0)),
        ],
        out_shape=[
            jax.ShapeDtypeStruct((2 * _N, _HH), jnp.float32),
            jax.ShapeDtypeStruct((_N, 1), jnp.float32),
            jax.ShapeDtypeStruct((1, _B), jnp.float32),
            jax.ShapeDtypeStruct((1, _B), jnp.float32),
        ],
    )(cent, seg2, cnt, oh, gn_w, gn_b, imp2, mna, mxa)


def _final_k1(hs, finalW, finalb, oh, imp0, imp3, mn, mx):
    def body(hlo, hhi, w_ref, b_ref, oh_ref, imp0_ref,
             imp3_ref, mn_ref, mx_ref, gsum_ref, s_ref, mnf_ref, mxf_ref):
        i = pl.program_id(0)
        h = jnp.concatenate([hlo[...], hhi[...]], axis=1)
        hf = jnp.dot(h, w_ref[...], preferred_element_type=jnp.float32) + b_ref[...]
        imp = _lazy_norm(imp3_ref[...], mn_ref[...], mx_ref[...], oh_ref[...])
        s = imp0_ref[...] + imp
        s_ref[...] = s
        oh = oh_ref[...]
        @pl.when(i == 0)
        def _():
            gsum_ref[...] = jnp.zeros((_B, _OUT), jnp.float32)
            mnf_ref[...] = jnp.full((1, _B), _BIG, jnp.float32)
            mxf_ref[...] = jnp.full((1, _B), -_BIG, jnp.float32)
        gsum_ref[...] += _segdot(oh, hf)
        mnf_ref[...] = jnp.minimum(
            mnf_ref[...],
            jnp.min(jnp.where(oh > 0, s, _BIG), axis=0, keepdims=True))
        mxf_ref[...] = jnp.maximum(
            mxf_ref[...],
            jnp.max(jnp.where(oh > 0, s, -_BIG), axis=0, keepdims=True))
    return pl.pallas_call(
        body,
        grid=(_NB,),
        in_specs=[
            pl.BlockSpec((_BN, _HH), lambda i: (i, 0)),
            pl.BlockSpec((_BN, _HH), lambda i: (_NB + i, 0)),
            pl.BlockSpec((_H, _OUT), lambda i: (0, 0)),
            pl.BlockSpec((1, _OUT), lambda i: (0, 0)),
            pl.BlockSpec((_BN, _B), lambda i: (i, 0)),
            pl.BlockSpec((_BN, 1), lambda i: (i, 0)),
            pl.BlockSpec((_BN, 1), lambda i: (i, 0)),
            pl.BlockSpec((1, _B), lambda i: (0, 0)),
            pl.BlockSpec((1, _B), lambda i: (0, 0)),
        ],
        out_specs=[
            pl.BlockSpec((_B, _OUT), lambda i: (0, 0)),
            pl.BlockSpec((_BN, 1), lambda i: (i, 0)),
            pl.BlockSpec((1, _B), lambda i: (0, 0)),
            pl.BlockSpec((1, _B), lambda i: (0, 0)),
        ],
        out_shape=[
            jax.ShapeDtypeStruct((_B, _OUT), jnp.float32),
            jax.ShapeDtypeStruct((_N, 1), jnp.float32),
            jax.ShapeDtypeStruct((1, _B), jnp.float32),
            jax.ShapeDtypeStruct((1, _B), jnp.float32),
        ],
    )(hs, hs, finalW, finalb, oh, imp0, imp3, mn, mx)


def _final_k2(gsum, cnt, s, oh, mnf, mxf):
    def body(gsum_ref, cnt_ref, s_ref, oh_ref, mn_ref, mx_ref,
             gemb_ref, fi_ref):
        gemb_ref[...] = gsum_ref[...] / cnt_ref[...]
        fi_ref[...] = _lazy_norm(s_ref[...], mn_ref[...], mx_ref[...],
                                 oh_ref[...])
    return pl.pallas_call(
        body,
        grid=(_NB,),
        in_specs=[
            pl.BlockSpec((_B, _OUT), lambda i: (0, 0)),
            pl.BlockSpec((_B, 1), lambda i: (0, 0)),
            pl.BlockSpec((_BN, 1), lambda i: (i, 0)),
            pl.BlockSpec((_BN, _B), lambda i: (i, 0)),
            pl.BlockSpec((1, _B), lambda i: (0, 0)),
            pl.BlockSpec((1, _B), lambda i: (0, 0)),
        ],
        out_specs=[
            pl.BlockSpec((_B, _OUT), lambda i: (0, 0)),
            pl.BlockSpec((_BN, 1), lambda i: (i, 0)),
        ],
        out_shape=[
            jax.ShapeDtypeStruct((_B, _OUT), jnp.float32),
            jax.ShapeDtypeStruct((_N, 1), jnp.float32),
        ],
    )(gsum, cnt, s, oh, mnf, mxf)


def kernel(x, edge_index, edge_attr, initial_importance, batch,
           imp_proj_W, imp_proj_b, init_W, init_b, conv_lin_W, conv_lin_b,
           edge_lin_W, edge_lin_b, gate_W, gate_b, prop_W, prop_b,
           gn_weight, gn_bias, gn_mean_scale, final_W, final_b):
    src = edge_index[0]
    dst = edge_index[1]
    imp0 = initial_importance[:, None]
    oh = (batch[:, None] == jnp.arange(_B, dtype=batch.dtype)[None, :]
          ).astype(jnp.float32)

    We = edge_lin_W.reshape(_L, _ED, 2, _HH).transpose(0, 2, 1, 3
                                                       ).reshape(2 * _L, _ED, _HH)
    be = edge_lin_b.reshape(_L, 2, 1, _HH).reshape(2 * _L, 1, _HH)
    e_all = _edge_proj(edge_attr, We, be)

    mn, mx, cnt_row = _seg_stats0(imp0, oh)
    cnt = cnt_row.reshape(_B, 1)
    hs = _init_h(x, imp0, mn, mx, oh,
                 imp_proj_W.reshape(1, _H), imp_proj_b.reshape(1, _H),
                 init_W[:_D], init_W[_D:], init_b.reshape(1, _H))
    imp_raw = imp0

    for l in range(_L):
        e_l = e_all[2 * l:2 * l + 2].reshape(2 * _E, _HH)
        aggs = _sc_agg(hs, e_l, src, dst)
        out, prop, seg = _layer_k1(
            hs, aggs, imp_raw, mn, mx, oh,
            conv_lin_W[l], conv_lin_b[l].reshape(1, _H),
            gate_W[l][:_H], gate_W[l][_H:].reshape(1, _H),
            gate_b[l].reshape(1, _H),
            prop_W[l].reshape(1, _H), prop_b[l].reshape(1, 1))
        cent, seg2, mna, mxa, imp2 = _layer_k2(
            out, seg, cnt, oh, gn_mean_scale[l].reshape(1, _H),
            imp_raw, mn, mx, prop)
        hs, imp_raw, mn, mx = _layer_k3(
            cent, seg2, cnt, oh, gn_weight[l].reshape(1, _H),
            gn_bias[l].reshape(1, _H), imp2, mna, mxa)

    gsum, s, mnf, mxf = _final_k1(hs, final_W, final_b.reshape(1, _OUT),
                                  oh, imp0, imp_raw, mn, mx)
    graph_emb, final_imp = _final_k2(gsum, cnt, s, oh, mnf, mxf)
    return (graph_emb, final_imp)


# speedup vs baseline: 2.9239x; 1.4563x over previous
import functools

import jax
import jax.numpy as jnp
from jax import lax
from jax.experimental import pallas as pl
from jax.experimental.pallas import tpu as pltpu
from jax.experimental.pallas import tpu_sc as plsc

_N = 10000
_E = 160000
_D = 256
_H = 256
_ED = 16
_L = 3
_OUT = 256
_B = 8
_DECAY = 0.9
_THRESH = 0.1
_HH = 128
_BIG = 3.0e38

_BN = 2000
_NB = _N // _BN
_BE = 4000
_NEB = _E // _BE

_NS = 16
_EPT = _E // _NS
_CH = 40
_NCH = _EPT // _CH
_RPT = 632
_NPAD = _NS * _RPT
_RLAST = _N - 15 * _RPT


def _lazy_norm(raw, mn_row, mx_row, oh):
    mnb = jnp.sum(oh * mn_row, axis=1, keepdims=True)
    mxb = jnp.sum(oh * mx_row, axis=1, keepdims=True)
    return (raw - mnb) / (mxb - mnb + 1e-8)


def _sigmoid(z):
    return 1.0 / (1.0 + jnp.exp(-z))


def _segdot(oh, x):
    return lax.dot_general(oh, x, (((0,), (0,)), ((), ())),
                           preferred_element_type=jnp.float32)


def _sc_agg(h_split, e_split, src, dst):
    mesh = plsc.VectorSubcoreMesh(core_axis_name="c", subcore_axis_name="s")

    @functools.partial(
        pl.kernel,
        out_type=jax.ShapeDtypeStruct((2 * _N, _HH), jnp.float32),
        mesh=mesh,
        scratch_types=[
            pltpu.VMEM((_EPT,), jnp.int32),
            pltpu.VMEM((_EPT,), jnp.int32),
            pltpu.VMEM((2, _CH, _HH), jnp.float32),
            pltpu.VMEM((2, _CH, _HH), jnp.float32),
            pltpu.VMEM((2, _CH, _HH), jnp.float32),
            pltpu.VMEM_SHARED((_N, _HH), jnp.float32),
            pltpu.SemaphoreType.DMA,
            pltpu.SemaphoreType.DMA,
            pltpu.SemaphoreType.DMA,
            pltpu.SemaphoreType.DMA,
            pltpu.SemaphoreType.DMA,
            pltpu.SemaphoreType.DMA,
        ],
    )
    def k(h_hbm, e_hbm, src_hbm, dst_hbm, agg_hbm,
          src_v, dst_v, rows_v, e_v, m_v, agg_sh,
          sg0, sg1, se0, se1, ss0, ss1):
        c = lax.axis_index("c")
        s = lax.axis_index("s")
        sgs = (sg0, sg1)
        ses = (se0, se1)
        sss = (ss0, ss1)

        def zf(i, _):
            def zg(j, _):
                m_v[0, i, pl.ds(j * 16, 16)] = jnp.zeros((16,), jnp.float32)
                return 0
            return lax.fori_loop(0, _HH // 16, zg, 0)
        lax.fori_loop(0, _CH, zf, 0)

        base_r = s * _RPT

        @pl.when(s < _NS - 1)
        def _():
            def zc(i, _):
                pltpu.sync_copy(m_v.at[0],
                                agg_sh.at[pl.ds(base_r + i * _CH, _CH)])
                return 0
            lax.fori_loop(0, _RPT // _CH, zc, 0)
            rem = _RPT - (_RPT // _CH) * _CH
            pltpu.sync_copy(m_v.at[0, pl.ds(0, rem)],
                            agg_sh.at[pl.ds(base_r + _RPT - rem, rem)])

        @pl.when(s == _NS - 1)
        def _():
            def zc(i, _):
                pltpu.sync_copy(m_v.at[0],
                                agg_sh.at[pl.ds(15 * _RPT + i * _CH, _CH)])
                return 0
            lax.fori_loop(0, _RLAST // _CH, zc, 0)
        plsc.subcore_barrier()

        cN = c * _N
        cE = c * _E
        ebase = s * _EPT

        pltpu.sync_copy(src_hbm.at[pl.ds(ebase, _EPT)], src_v)
        pltpu.sync_copy(dst_hbm.at[pl.ds(ebase, _EPT)], dst_v)
        def off(i, _):
            src_v[pl.ds(i * 16, 16)] = src_v[pl.ds(i * 16, 16)] + cN
            return 0
        lax.fori_loop(0, _EPT // 16, off, 0)

        def start(kk, b):
            pltpu.async_copy(h_hbm.at[src_v.at[pl.ds(kk * _CH, _CH)]],
                             rows_v.at[b], sgs[b])
            pltpu.async_copy(e_hbm.at[pl.ds(cE + ebase + kk * _CH, _CH)],
                             e_v.at[b], ses[b])

        start(0, 0)
        start(1, 1)

        def outer(g, _):
            for b in (0, 1):
                kk = 2 * g + b
                pltpu.make_async_copy(
                    h_hbm.at[src_v.at[pl.ds(kk * _CH, _CH)]],
                    rows_v.at[b], sgs[b]).wait()
                pltpu.make_async_copy(
                    e_hbm.at[pl.ds(cE + ebase + kk * _CH, _CH)],
                    e_v.at[b], ses[b]).wait()

                @pl.when(g >= 1)
                def _():
                    pltpu.make_async_copy(
                        m_v.at[b],
                        agg_sh.at[dst_v.at[pl.ds((kk - 2) * _CH, _CH)]],
                        sss[b]).wait()

                def fc(i, _):
                    for j in range(_HH // 16):
                        o = j * 16
                        m_v[b, i, pl.ds(o, 16)] = jnp.maximum(
                            rows_v[b, i, pl.ds(o, 16)]
                            + e_v[b, i, pl.ds(o, 16)], 0.0)
                    return 0
                lax.fori_loop(0, _CH, fc, 0)

                pltpu.async_copy(
                    m_v.at[b],
                    agg_sh.at[dst_v.at[pl.ds(kk * _CH, _CH)]],
                    sss[b], add=True)

                @pl.when(kk + 2 < _NCH)
                def _():
                    start(kk + 2, b)
            return 0
        lax.fori_loop(0, _NCH // 2, outer, 0)

        pltpu.make_async_copy(
            m_v.at[0], agg_sh.at[dst_v.at[pl.ds((_NCH - 2) * _CH, _CH)]],
            ss0).wait()
        pltpu.make_async_copy(
            m_v.at[1], agg_sh.at[dst_v.at[pl.ds((_NCH - 1) * _CH, _CH)]],
            ss1).wait()
        plsc.subcore_barrier()

        @pl.when(s < _NS - 1)
        def _():
            pltpu.sync_copy(agg_sh.at[pl.ds(s * _RPT, _RPT)],
                            agg_hbm.at[pl.ds(cN + s * _RPT, _RPT)])

        @pl.when(s == _NS - 1)
        def _():
            pltpu.sync_copy(agg_sh.at[pl.ds(15 * _RPT, _RLAST)],
                            agg_hbm.at[pl.ds(cN + 15 * _RPT, _RLAST)])

    return k(h_split, e_split, src, dst)


def _edge_proj(edge_attr, We, be):
    def body(ea_ref, w_ref, b_ref, out_ref):
        out_ref[...] = (jnp.dot(ea_ref[...], w_ref[0],
                                preferred_element_type=jnp.float32)
                        + b_ref[0])[None]
    return pl.pallas_call(
        body,
        grid=(2 * _L, _NEB),
        in_specs=[
            pl.BlockSpec((_BE, _ED), lambda p, i: (i, 0)),
            pl.BlockSpec((1, _ED, _HH), lambda p, i: (p, 0, 0)),
            pl.BlockSpec((1, 1, _HH), lambda p, i: (p, 0, 0)),
        ],
        out_specs=pl.BlockSpec((1, _BE, _HH), lambda p, i: (p, i, 0)),
        out_shape=jax.ShapeDtypeStruct((2 * _L, _E, _HH), jnp.float32),
    )(edge_attr, We, be)


def _seg_stats0(imp0, oh):
    def body(imp_ref, oh_ref, mn_ref, mx_ref, cnt_ref):
        i = pl.program_id(0)
        @pl.when(i == 0)
        def _():
            mn_ref[...] = jnp.full((1, _B), _BIG, jnp.float32)
            mx_ref[...] = jnp.full((1, _B), -_BIG, jnp.float32)
            cnt_ref[...] = jnp.zeros((1, _B), jnp.float32)
        imp = imp_ref[...]
        oh = oh_ref[...]
        mn_ref[...] = jnp.minimum(
            mn_ref[...],
            jnp.min(jnp.where(oh > 0, imp, _BIG), axis=0, keepdims=True))
        mx_ref[...] = jnp.maximum(
            mx_ref[...],
            jnp.max(jnp.where(oh > 0, imp, -_BIG), axis=0, keepdims=True))
        cnt_ref[...] += jnp.sum(oh, axis=0, keepdims=True)
    return pl.pallas_call(
        body,
        grid=(_NB,),
        in_specs=[
            pl.BlockSpec((_BN, 1), lambda i: (i, 0)),
            pl.BlockSpec((_BN, _B), lambda i: (i, 0)),
        ],
        out_specs=[
            pl.BlockSpec((1, _B), lambda i: (0, 0)),
            pl.BlockSpec((1, _B), lambda i: (0, 0)),
            pl.BlockSpec((1, _B), lambda i: (0, 0)),
        ],
        out_shape=[
            jax.ShapeDtypeStruct((1, _B), jnp.float32),
            jax.ShapeDtypeStruct((1, _B), jnp.float32),
            jax.ShapeDtypeStruct((1, _B), jnp.float32),
        ],
    )(imp0, oh)


def _init_h(x, imp0, mn0, mx0, oh, projW, projb, Wx, Wi, initb):
    def body(x_ref, imp_ref, mn_ref, mx_ref, oh_ref, pw_ref, pb_ref,
             wx_ref, wi_ref, b_ref, out_ref):
        imp = _lazy_norm(imp_ref[...], mn_ref[...], mx_ref[...], oh_ref[...])
        emb = imp * pw_ref[...] + pb_ref[...]
        out_ref[...] = (jnp.dot(x_ref[...], wx_ref[...],
                                preferred_element_type=jnp.float32)
                        + jnp.dot(emb, wi_ref[...],
                                  preferred_element_type=jnp.float32)
                        + b_ref[...])
    return pl.pallas_call(
        body,
        grid=(2, _NB),
        in_specs=[
            pl.BlockSpec((_BN, _D), lambda c, i: (i, 0)),
            pl.BlockSpec((_BN, 1), lambda c, i: (i, 0)),
            pl.BlockSpec((1, _B), lambda c, i: (0, 0)),
            pl.BlockSpec((1, _B), lambda c, i: (0, 0)),
            pl.BlockSpec((_BN, _B), lambda c, i: (i, 0)),
            pl.BlockSpec((1, _H), lambda c, i: (0, 0)),
            pl.BlockSpec((1, _H), lambda c, i: (0, 0)),
            pl.BlockSpec((_D, _HH), lambda c, i: (0, c)),
            pl.BlockSpec((_H, _HH), lambda c, i: (0, c)),
            pl.BlockSpec((1, _HH), lambda c, i: (0, c)),
        ],
        out_specs=pl.BlockSpec((_BN, _HH), lambda c, i: (c * _NB + i, 0)),
        out_shape=jax.ShapeDtypeStruct((2 * _N, _HH), jnp.float32),
    )(x, imp0, mn0, mx0, oh, projW, projb, Wx, Wi, initb)


def _layer_k1(hs, aggs, imp_raw, mn, mx, oh,
              convW, convb, gWh, gWi, gb, propWr, propb):
    def body(hlo, hhi, alo, ahi, imp_ref, mn_ref, mx_ref, oh_ref,
             cw, cb, gwh, gwi, gbr, pwr, pbr, out_ref, prop_ref, seg_ref):
        i = pl.program_id(0)
        h = jnp.concatenate([hlo[...], hhi[...]], axis=1)
        agg = jnp.concatenate([alo[...], ahi[...]], axis=1)
        imp = _lazy_norm(imp_ref[...], mn_ref[...], mx_ref[...], oh_ref[...])
        conv = jnp.dot(h + agg, cw[...],
                       preferred_element_type=jnp.float32) + cb[...]
        gate = _sigmoid(jnp.dot(conv, gwh[...],
                                preferred_element_type=jnp.float32)
                        + imp * gwi[...] + gbr[...])
        o = gate * conv + (1.0 - gate) * h
        out_ref[...] = o
        prop_ref[...] = jnp.sum(o * pwr[...], axis=1, keepdims=True) + pbr[...]
        @pl.when(i == 0)
        def _():
            seg_ref[...] = jnp.zeros((_B, _H), jnp.float32)
        seg_ref[...] += _segdot(oh_ref[...], o)
    return pl.pallas_call(
        body,
        grid=(_NB,),
        in_specs=[
            pl.BlockSpec((_BN, _HH), lambda i: (i, 0)),
            pl.BlockSpec((_BN, _HH), lambda i: (_NB + i, 0)),
            pl.BlockSpec((_BN, _HH), lambda i: (i, 0)),
            pl.BlockSpec((_BN, _HH), lambda i: (_NB + i, 0)),
            pl.BlockSpec((_BN, 1), lambda i: (i, 0)),
            pl.BlockSpec((1, _B), lambda i: (0, 0)),
            pl.BlockSpec((1, _B), lambda i: (0, 0)),
            pl.BlockSpec((_BN, _B), lambda i: (i, 0)),
            pl.BlockSpec((_H, _H), lambda i: (0, 0)),
            pl.BlockSpec((1, _H), lambda i: (0, 0)),
            pl.BlockSpec((_H, _H), lambda i: (0, 0)),
            pl.BlockSpec((1, _H), lambda i: (0, 0)),
            pl.BlockSpec((1, _H), lambda i: (0, 0)),
            pl.BlockSpec((1, _H), lambda i: (0, 0)),
            pl.BlockSpec((1, 1), lambda i: (0, 0)),
        ],
        out_specs=[
            pl.BlockSpec((_BN, _H), lambda i: (i, 0)),
            pl.BlockSpec((_BN, 1), lambda i: (i, 0)),
            pl.BlockSpec((_B, _H), lambda i: (0, 0)),
        ],
        out_shape=[
            jax.ShapeDtypeStruct((_N, _H), jnp.float32),
            jax.ShapeDtypeStruct((_N, 1), jnp.float32),
            jax.ShapeDtypeStruct((_B, _H), jnp.float32),
        ],
    )(hs, hs, aggs, aggs, imp_raw, mn, mx, oh,
      convW, convb, gWh, gWi, gb, propWr, propb)


def _layer_k2(out, seg, cnt, oh, gn_ms, imp_raw, mn, mx, prop):
    def body(out_ref, seg_ref, cnt_ref, oh_ref, ms_ref,
             imp_ref, mn_ref, mx_ref, prop_ref,
             cent_ref, seg2_ref, mna_ref, mxa_ref, imp2_ref):
        i = pl.program_id(0)
        mean = seg_ref[...] / cnt_ref[...]
        meanb = jnp.dot(oh_ref[...], mean, preferred_element_type=jnp.float32)
        cent = out_ref[...] - meanb * ms_ref[...]
        cent_ref[...] = cent
        imp = _lazy_norm(imp_ref[...], mn_ref[...], mx_ref[...], oh_ref[...])
        imp2 = _DECAY * imp + (1.0 - _DECAY) * prop_ref[...]
        imp2_ref[...] = imp2
        oh = oh_ref[...]
        @pl.when(i == 0)
        def _():
            seg2_ref[...] = jnp.zeros((_B, _H), jnp.float32)
            mna_ref[...] = jnp.full((1, _B), _BIG, jnp.float32)
            mxa_ref[...] = jnp.full((1, _B), -_BIG, jnp.float32)
        seg2_ref[...] += _segdot(oh, cent * cent)
        mna_ref[...] = jnp.minimum(
            mna_ref[...],
            jnp.min(jnp.where(oh > 0, imp2, _BIG), axis=0, keepdims=True))
        mxa_ref[...] = jnp.maximum(
            mxa_ref[...],
            jnp.max(jnp.where(oh > 0, imp2, -_BIG), axis=0, keepdims=True))
    return pl.pallas_call(
        body,
        grid=(_NB,),
        in_specs=[
            pl.BlockSpec((_BN, _H), lambda i: (i, 0)),
            pl.BlockSpec((_B, _H), lambda i: (0, 0)),
            pl.BlockSpec((_B, 1), lambda i: (0, 0)),
            pl.BlockSpec((_BN, _B), lambda i: (i, 0)),
            pl.BlockSpec((1, _H), lambda i: (0, 0)),
            pl.BlockSpec((_BN, 1), lambda i: (i, 0)),
            pl.BlockSpec((1, _B), lambda i: (0, 0)),
            pl.BlockSpec((1, _B), lambda i: (0, 0)),
            pl.BlockSpec((_BN, 1), lambda i: (i, 0)),
        ],
        out_specs=[
            pl.BlockSpec((_BN, _H), lambda i: (i, 0)),
            pl.BlockSpec((_B, _H), lambda i: (0, 0)),
            pl.BlockSpec((1, _B), lambda i: (0, 0)),
            pl.BlockSpec((1, _B), lambda i: (0, 0)),
            pl.BlockSpec((_BN, 1), lambda i: (i, 0)),
        ],
        out_shape=[
            jax.ShapeDtypeStruct((_N, _H), jnp.float32),
            jax.ShapeDtypeStruct((_B, _H), jnp.float32),
            jax.ShapeDtypeStruct((1, _B), jnp.float32),
            jax.ShapeDtypeStruct((1, _B), jnp.float32),
            jax.ShapeDtypeStruct((_N, 1), jnp.float32),
        ],
    )(out, seg, cnt, oh, gn_ms, imp_raw, mn, mx, prop)


def _layer_k3(cent, seg2, cnt, oh, gn_w, gn_b, imp2, mna, mxa):
    def body(cent_ref, seg2_ref, cnt_ref, oh_ref, gw_ref, gb_ref,
             imp2_ref, mna_ref, mxa_ref,
             hs_ref, imp3_ref, mnb_ref, mxb_ref):
        c = pl.program_id(0)
        i = pl.program_id(1)
        var = seg2_ref[...] / cnt_ref[...]
        varb = jnp.dot(oh_ref[...], var, preferred_element_type=jnp.float32)
        normed = cent_ref[...] * lax.rsqrt(varb + 1e-5) * gw_ref[...] + gb_ref[...]
        hs_ref[...] = jnp.maximum(normed, 0.0)
        imp2n = _lazy_norm(imp2_ref[...], mna_ref[...], mxa_ref[...], oh_ref[...])
        imp3 = jnp.where(imp2n < _THRESH, 0.0, imp2n)
        imp3_ref[...] = imp3
        oh = oh_ref[...]
        @pl.when((c == 0) & (i == 0))
        def _():
            mnb_ref[...] = jnp.full((1, _B), _BIG, jnp.float32)
            mxb_ref[...] = jnp.full((1, _B), -_BIG, jnp.float32)
        mnb_ref[...] = jnp.minimum(
            mnb_ref[...],
            jnp.min(jnp.where(oh > 0, imp3, _BIG), axis=0, keepdims=True))
        mxb_ref[...] = jnp.maximum(
            mxb_ref[...],
            jnp.max(jnp.where(oh > 0, imp3, -_BIG), axis=0, keepdims=True))
    return pl.pallas_call(
        body,
        grid=(2, _NB),
        in_specs=[
            pl.BlockSpec((_BN, _HH), lambda c, i: (i, c)),
            pl.BlockSpec((_B, _HH), lambda c, i: (0, c)),
            pl.BlockSpec((_B, 1), lambda c, i: (0, 0)),
            pl.BlockSpec((_BN, _B), lambda c, i: (i, 0)),
            pl.BlockSpec((1, _HH), lambda c, i: (0, c)),
            pl.BlockSpec((1, _HH), lambda c, i: (0, c)),
            pl.BlockSpec((_BN, 1), lambda c, i: (i, 0)),
            pl.BlockSpec((1, _B), lambda c, i: (0, 0)),
            pl.BlockSpec((1, _B), lambda c, i: (0, 0)),
        ],
        out_specs=[
            pl.BlockSpec((_BN, _HH), lambda c, i: (c * _NB + i, 0)),
            pl.BlockSpec((_BN, 1), lambda c, i: (i, 0)),
            pl.BlockSpec((1, _B), lambda c, i: (0, 0)),
            pl.BlockSpec((1, _B), lambda c, i: (0, 0)),
        ],
        out_shape=[
            jax.ShapeDtypeStruct((2 * _N, _HH), jnp.float32),
            jax.ShapeDtypeStruct((_N, 1), jnp.float32),
            jax.ShapeDtypeStruct((1, _B), jnp.float32),
            jax.ShapeDtypeStruct((1, _B), jnp.float32),
        ],
    )(cent, seg2, cnt, oh, gn_w, gn_b, imp2, mna, mxa)


def _final_k1(hs, finalW, finalb, oh, imp0, imp3, mn, mx):
    def body(hlo, hhi, w_ref, b_ref, oh_ref, imp0_ref,
             imp3_ref, mn_ref, mx_ref, gsum_ref, s_ref, mnf_ref, mxf_ref):
        i = pl.program_id(0)
        h = jnp.concatenate([hlo[...], hhi[...]], axis=1)
        hf = jnp.dot(h, w_ref[...], preferred_element_type=jnp.float32) + b_ref[...]
        imp = _lazy_norm(imp3_ref[...], mn_ref[...], mx_ref[...], oh_ref[...])
        s = imp0_ref[...] + imp
        s_ref[...] = s
        oh = oh_ref[...]
        @pl.when(i == 0)
        def _():
            gsum_ref[...] = jnp.zeros((_B, _OUT), jnp.float32)
            mnf_ref[...] = jnp.full((1, _B), _BIG, jnp.float32)
            mxf_ref[...] = jnp.full((1, _B), -_BIG, jnp.float32)
        gsum_ref[...] += _segdot(oh, hf)
        mnf_ref[...] = jnp.minimum(
            mnf_ref[...],
            jnp.min(jnp.where(oh > 0, s, _BIG), axis=0, keepdims=True))
        mxf_ref[...] = jnp.maximum(
            mxf_ref[...],
            jnp.max(jnp.where(oh > 0, s, -_BIG), axis=0, keepdims=True))
    return pl.pallas_call(
        body,
        grid=(_NB,),
        in_specs=[
            pl.BlockSpec((_BN, _HH), lambda i: (i, 0)),
            pl.BlockSpec((_BN, _HH), lambda i: (_NB + i, 0)),
            pl.BlockSpec((_H, _OUT), lambda i: (0, 0)),
            pl.BlockSpec((1, _OUT), lambda i: (0, 0)),
            pl.BlockSpec((_BN, _B), lambda i: (i, 0)),
            pl.BlockSpec((_BN, 1), lambda i: (i, 0)),
            pl.BlockSpec((_BN, 1), lambda i: (i, 0)),
            pl.BlockSpec((1, _B), lambda i: (0, 0)),
            pl.BlockSpec((1, _B), lambda i: (0, 0)),
        ],
        out_specs=[
            pl.BlockSpec((_B, _OUT), lambda i: (0, 0)),
            pl.BlockSpec((_BN, 1), lambda i: (i, 0)),
            pl.BlockSpec((1, _B), lambda i: (0, 0)),
            pl.BlockSpec((1, _B), lambda i: (0, 0)),
        ],
        out_shape=[
            jax.ShapeDtypeStruct((_B, _OUT), jnp.float32),
            jax.ShapeDtypeStruct((_N, 1), jnp.float32),
            jax.ShapeDtypeStruct((1, _B), jnp.float32),
            jax.ShapeDtypeStruct((1, _B), jnp.float32),
        ],
    )(hs, hs, finalW, finalb, oh, imp0, imp3, mn, mx)


def _final_k2(gsum, cnt, s, oh, mnf, mxf):
    def body(gsum_ref, cnt_ref, s_ref, oh_ref, mn_ref, mx_ref,
             gemb_ref, fi_ref):
        gemb_ref[...] = gsum_ref[...] / cnt_ref[...]
        fi_ref[...] = _lazy_norm(s_ref[...], mn_ref[...], mx_ref[...],
                                 oh_ref[...])
    return pl.pallas_call(
        body,
        grid=(_NB,),
        in_specs=[
            pl.BlockSpec((_B, _OUT), lambda i: (0, 0)),
            pl.BlockSpec((_B, 1), lambda i: (0, 0)),
            pl.BlockSpec((_BN, 1), lambda i: (i, 0)),
            pl.BlockSpec((_BN, _B), lambda i: (i, 0)),
            pl.BlockSpec((1, _B), lambda i: (0, 0)),
            pl.BlockSpec((1, _B), lambda i: (0, 0)),
        ],
        out_specs=[
            pl.BlockSpec((_B, _OUT), lambda i: (0, 0)),
            pl.BlockSpec((_BN, 1), lambda i: (i, 0)),
        ],
        out_shape=[
            jax.ShapeDtypeStruct((_B, _OUT), jnp.float32),
            jax.ShapeDtypeStruct((_N, 1), jnp.float32),
        ],
    )(gsum, cnt, s, oh, mnf, mxf)


def kernel(x, edge_index, edge_attr, initial_importance, batch,
           imp_proj_W, imp_proj_b, init_W, init_b, conv_lin_W, conv_lin_b,
           edge_lin_W, edge_lin_b, gate_W, gate_b, prop_W, prop_b,
           gn_weight, gn_bias, gn_mean_scale, final_W, final_b):
    src = edge_index[0]
    dst = edge_index[1]
    imp0 = initial_importance[:, None]
    oh = (batch[:, None] == jnp.arange(_B, dtype=batch.dtype)[None, :]
          ).astype(jnp.float32)

    We = edge_lin_W.reshape(_L, _ED, 2, _HH).transpose(0, 2, 1, 3
                                                       ).reshape(2 * _L, _ED, _HH)
    be = edge_lin_b.reshape(_L, 2, 1, _HH).reshape(2 * _L, 1, _HH)
    e_all = _edge_proj(edge_attr, We, be)

    mn, mx, cnt_row = _seg_stats0(imp0, oh)
    cnt = cnt_row.reshape(_B, 1)
    hs = _init_h(x, imp0, mn, mx, oh,
                 imp_proj_W.reshape(1, _H), imp_proj_b.reshape(1, _H),
                 init_W[:_D], init_W[_D:], init_b.reshape(1, _H))
    imp_raw = imp0

    for l in range(_L):
        e_l = e_all[2 * l:2 * l + 2].reshape(2 * _E, _HH)
        aggs = _sc_agg(hs, e_l, src, dst)
        out, prop, seg = _layer_k1(
            hs, aggs, imp_raw, mn, mx, oh,
            conv_lin_W[l], conv_lin_b[l].reshape(1, _H),
            gate_W[l][:_H], gate_W[l][_H:].reshape(1, _H),
            gate_b[l].reshape(1, _H),
            prop_W[l].reshape(1, _H), prop_b[l].reshape(1, 1))
        cent, seg2, mna, mxa, imp2 = _layer_k2(
            out, seg, cnt, oh, gn_mean_scale[l].reshape(1, _H),
            imp_raw, mn, mx, prop)
        hs, imp_raw, mn, mx = _layer_k3(
            cent, seg2, cnt, oh, gn_weight[l].reshape(1, _H),
            gn_bias[l].reshape(1, _H), imp2, mna, mxa)

    gsum, s, mnf, mxf = _final_k1(hs, final_W, final_b.reshape(1, _OUT),
                                  oh, imp0, imp_raw, mn, mx)
    graph_emb, final_imp = _final_k2(gsum, cnt, s, oh, mnf, mxf)
    return (graph_emb, final_imp)


# speedup vs baseline: 4.1537x; 1.4206x over previous
import functools

import jax
import jax.numpy as jnp
from jax import lax
from jax.experimental import pallas as pl
from jax.experimental.pallas import tpu as pltpu
from jax.experimental.pallas import tpu_sc as plsc

_N = 10000
_E = 160000
_D = 256
_H = 256
_ED = 16
_L = 3
_OUT = 256
_B = 8
_DECAY = 0.9
_THRESH = 0.1
_HH = 128
_BIG = 3.0e38

_BN = 2000
_NB = _N // _BN
_BE = 4000
_NEB = _E // _BE

_NS = 16
_EPT = _E // _NS
_CH = 40
_NCH = _EPT // _CH
_RPT = 632
_NPAD = _NS * _RPT
_RLAST = _N - 15 * _RPT


def _lazy_norm(raw, mn_row, mx_row, oh):
    mnb = jnp.sum(oh * mn_row, axis=1, keepdims=True)
    mxb = jnp.sum(oh * mx_row, axis=1, keepdims=True)
    return (raw - mnb) / (mxb - mnb + 1e-8)


def _sigmoid(z):
    return 1.0 / (1.0 + jnp.exp(-z))


def _segdot(oh, x):
    return lax.dot_general(oh, x, (((0,), (0,)), ((), ())),
                           preferred_element_type=jnp.float32)


def _sc_agg(h_split, e_split, src, dst):
    mesh = plsc.VectorSubcoreMesh(core_axis_name="c", subcore_axis_name="s")

    @functools.partial(
        pl.kernel,
        out_type=jax.ShapeDtypeStruct((2 * _N, _HH), jnp.float32),
        mesh=mesh,
        scratch_types=[
            pltpu.VMEM((_EPT,), jnp.int32),
            pltpu.VMEM((_EPT,), jnp.int32),
            pltpu.VMEM((2, _CH, _HH), jnp.float32),
            pltpu.VMEM((2, _CH, _HH), jnp.float32),
            pltpu.VMEM((2, _CH, _HH), jnp.float32),
            pltpu.VMEM_SHARED((_N, _HH), jnp.float32),
            pltpu.SemaphoreType.DMA,
            pltpu.SemaphoreType.DMA,
            pltpu.SemaphoreType.DMA,
            pltpu.SemaphoreType.DMA,
            pltpu.SemaphoreType.DMA,
            pltpu.SemaphoreType.DMA,
        ],
    )
    def k(h_hbm, e_hbm, src_hbm, dst_hbm, agg_hbm,
          src_v, dst_v, rows_v, e_v, m_v, agg_sh,
          sg0, sg1, se0, se1, ss0, ss1):
        c = lax.axis_index("c")
        s = lax.axis_index("s")
        sgs = (sg0, sg1)
        ses = (se0, se1)
        sss = (ss0, ss1)

        def zf(i, _):
            def zg(j, _):
                m_v[0, i, pl.ds(j * 16, 16)] = jnp.zeros((16,), jnp.float32)
                return 0
            return lax.fori_loop(0, _HH // 16, zg, 0)
        lax.fori_loop(0, _CH, zf, 0)

        base_r = s * _RPT

        @pl.when(s < _NS - 1)
        def _():
            def zc(i, _):
                pltpu.sync_copy(m_v.at[0],
                                agg_sh.at[pl.ds(base_r + i * _CH, _CH)])
                return 0
            lax.fori_loop(0, _RPT // _CH, zc, 0)
            rem = _RPT - (_RPT // _CH) * _CH
            pltpu.sync_copy(m_v.at[0, pl.ds(0, rem)],
                            agg_sh.at[pl.ds(base_r + _RPT - rem, rem)])

        @pl.when(s == _NS - 1)
        def _():
            def zc(i, _):
                pltpu.sync_copy(m_v.at[0],
                                agg_sh.at[pl.ds(15 * _RPT + i * _CH, _CH)])
                return 0
            lax.fori_loop(0, _RLAST // _CH, zc, 0)
        plsc.subcore_barrier()

        cN = c * _N
        cE = c * _E
        ebase = s * _EPT

        pltpu.sync_copy(src_hbm.at[pl.ds(ebase, _EPT)], src_v)
        pltpu.sync_copy(dst_hbm.at[pl.ds(ebase, _EPT)], dst_v)
        def off(i, _):
            src_v[pl.ds(i * 16, 16)] = src_v[pl.ds(i * 16, 16)] + cN
            return 0
        lax.fori_loop(0, _EPT // 16, off, 0)

        def start(kk, b):
            pltpu.async_copy(h_hbm.at[src_v.at[pl.ds(kk * _CH, _CH)]],
                             rows_v.at[b], sgs[b])
            pltpu.async_copy(e_hbm.at[pl.ds(cE + ebase + kk * _CH, _CH)],
                             e_v.at[b], ses[b])

        start(0, 0)
        start(1, 1)

        def outer(g, _):
            for b in (0, 1):
                kk = 2 * g + b
                pltpu.make_async_copy(
                    h_hbm.at[src_v.at[pl.ds(kk * _CH, _CH)]],
                    rows_v.at[b], sgs[b]).wait()
                pltpu.make_async_copy(
                    e_hbm.at[pl.ds(cE + ebase + kk * _CH, _CH)],
                    e_v.at[b], ses[b]).wait()

                @pl.when(g >= 1)
                def _():
                    pltpu.make_async_copy(
                        m_v.at[b],
                        agg_sh.at[dst_v.at[pl.ds((kk - 2) * _CH, _CH)]],
                        sss[b]).wait()

                def fc(i, _):
                    for j in range(_HH // 16):
                        o = j * 16
                        m_v[b, i, pl.ds(o, 16)] = jnp.maximum(
                            rows_v[b, i, pl.ds(o, 16)]
                            + e_v[b, i, pl.ds(o, 16)], 0.0)
                    return 0
                lax.fori_loop(0, _CH, fc, 0)

                pltpu.async_copy(
                    m_v.at[b],
                    agg_sh.at[dst_v.at[pl.ds(kk * _CH, _CH)]],
                    sss[b], add=True)

                @pl.when(kk + 2 < _NCH)
                def _():
                    start(kk + 2, b)
            return 0
        lax.fori_loop(0, _NCH // 2, outer, 0)

        pltpu.make_async_copy(
            m_v.at[0], agg_sh.at[dst_v.at[pl.ds((_NCH - 2) * _CH, _CH)]],
            ss0).wait()
        pltpu.make_async_copy(
            m_v.at[1], agg_sh.at[dst_v.at[pl.ds((_NCH - 1) * _CH, _CH)]],
            ss1).wait()
        plsc.subcore_barrier()

        @pl.when(s < _NS - 1)
        def _():
            pltpu.sync_copy(agg_sh.at[pl.ds(s * _RPT, _RPT)],
                            agg_hbm.at[pl.ds(cN + s * _RPT, _RPT)])

        @pl.when(s == _NS - 1)
        def _():
            pltpu.sync_copy(agg_sh.at[pl.ds(15 * _RPT, _RLAST)],
                            agg_hbm.at[pl.ds(cN + 15 * _RPT, _RLAST)])

    return k(h_split, e_split, src, dst)


def _edge_proj(edge_attr, We, be):
    def body(ea_ref, w_ref, b_ref, out_ref):
        out_ref[...] = (jnp.dot(ea_ref[...], w_ref[0],
                                preferred_element_type=jnp.float32)
                        + b_ref[0])[None]
    return pl.pallas_call(
        body,
        grid=(2, _NEB),
        in_specs=[
            pl.BlockSpec((_BE, _ED), lambda p, i: (i, 0)),
            pl.BlockSpec((1, _ED, _HH), lambda p, i: (p, 0, 0)),
            pl.BlockSpec((1, 1, _HH), lambda p, i: (p, 0, 0)),
        ],
        out_specs=pl.BlockSpec((1, _BE, _HH), lambda p, i: (p, i, 0)),
        out_shape=jax.ShapeDtypeStruct((2, _E, _HH), jnp.float32),
    )(edge_attr, We, be)


def _seg_stats0(imp0, oh):
    def body(imp_ref, oh_ref, mn_ref, mx_ref, cnt_ref):
        i = pl.program_id(0)
        @pl.when(i == 0)
        def _():
            mn_ref[...] = jnp.full((1, _B), _BIG, jnp.float32)
            mx_ref[...] = jnp.full((1, _B), -_BIG, jnp.float32)
            cnt_ref[...] = jnp.zeros((1, _B), jnp.float32)
        imp = imp_ref[...]
        oh = oh_ref[...]
        mn_ref[...] = jnp.minimum(
            mn_ref[...],
            jnp.min(jnp.where(oh > 0, imp, _BIG), axis=0, keepdims=True))
        mx_ref[...] = jnp.maximum(
            mx_ref[...],
            jnp.max(jnp.where(oh > 0, imp, -_BIG), axis=0, keepdims=True))
        cnt_ref[...] += jnp.sum(oh, axis=0, keepdims=True)
    return pl.pallas_call(
        body,
        grid=(_NB,),
        in_specs=[
            pl.BlockSpec((_BN, 1), lambda i: (i, 0)),
            pl.BlockSpec((_BN, _B), lambda i: (i, 0)),
        ],
        out_specs=[
            pl.BlockSpec((1, _B), lambda i: (0, 0)),
            pl.BlockSpec((1, _B), lambda i: (0, 0)),
            pl.BlockSpec((1, _B), lambda i: (0, 0)),
        ],
        out_shape=[
            jax.ShapeDtypeStruct((1, _B), jnp.float32),
            jax.ShapeDtypeStruct((1, _B), jnp.float32),
            jax.ShapeDtypeStruct((1, _B), jnp.float32),
        ],
    )(imp0, oh)


def _init_h(x, imp0, mn0, mx0, oh, projW, projb, Wx, Wi, initb):
    def body(x_ref, imp_ref, mn_ref, mx_ref, oh_ref, pw_ref, pb_ref,
             wx_ref, wi_ref, b_ref, out_ref):
        imp = _lazy_norm(imp_ref[...], mn_ref[...], mx_ref[...], oh_ref[...])
        emb = imp * pw_ref[...] + pb_ref[...]
        out_ref[...] = (jnp.dot(x_ref[...], wx_ref[...],
                                preferred_element_type=jnp.float32)
                        + jnp.dot(emb, wi_ref[...],
                                  preferred_element_type=jnp.float32)
                        + b_ref[...])
    return pl.pallas_call(
        body,
        grid=(2, _NB),
        in_specs=[
            pl.BlockSpec((_BN, _D), lambda c, i: (i, 0)),
            pl.BlockSpec((_BN, 1), lambda c, i: (i, 0)),
            pl.BlockSpec((1, _B), lambda c, i: (0, 0)),
            pl.BlockSpec((1, _B), lambda c, i: (0, 0)),
            pl.BlockSpec((_BN, _B), lambda c, i: (i, 0)),
            pl.BlockSpec((1, _H), lambda c, i: (0, 0)),
            pl.BlockSpec((1, _H), lambda c, i: (0, 0)),
            pl.BlockSpec((_D, _HH), lambda c, i: (0, c)),
            pl.BlockSpec((_H, _HH), lambda c, i: (0, c)),
            pl.BlockSpec((1, _HH), lambda c, i: (0, c)),
        ],
        out_specs=pl.BlockSpec((_BN, _HH), lambda c, i: (c * _NB + i, 0)),
        out_shape=jax.ShapeDtypeStruct((2 * _N, _HH), jnp.float32),
    )(x, imp0, mn0, mx0, oh, projW, projb, Wx, Wi, initb)


def _layer_k1(hs, aggs, imp_raw, mn, mx, oh,
              convW, convb, gWh, gWi, gb, propWr, propb):
    def body(hlo, hhi, alo, ahi, imp_ref, mn_ref, mx_ref, oh_ref,
             cw, cb, gwh, gwi, gbr, pwr, pbr,
             out_ref, imp2_ref, seg_ref, segq_ref, mna_ref, mxa_ref):
        i = pl.program_id(0)
        h = jnp.concatenate([hlo[...], hhi[...]], axis=1)
        agg = jnp.concatenate([alo[...], ahi[...]], axis=1)
        imp = _lazy_norm(imp_ref[...], mn_ref[...], mx_ref[...], oh_ref[...])
        conv = jnp.dot(h + agg, cw[...],
                       preferred_element_type=jnp.float32) + cb[...]
        gate = _sigmoid(jnp.dot(conv, gwh[...],
                                preferred_element_type=jnp.float32)
                        + imp * gwi[...] + gbr[...])
        o = gate * conv + (1.0 - gate) * h
        out_ref[...] = o
        prop = jnp.sum(o * pwr[...], axis=1, keepdims=True) + pbr[...]
        imp2 = _DECAY * imp + (1.0 - _DECAY) * prop
        imp2_ref[...] = imp2
        oh = oh_ref[...]
        @pl.when(i == 0)
        def _():
            seg_ref[...] = jnp.zeros((_B, _H), jnp.float32)
            segq_ref[...] = jnp.zeros((_B, _H), jnp.float32)
            mna_ref[...] = jnp.full((1, _B), _BIG, jnp.float32)
            mxa_ref[...] = jnp.full((1, _B), -_BIG, jnp.float32)
        seg_ref[...] += _segdot(oh, o)
        segq_ref[...] += _segdot(oh, o * o)
        mna_ref[...] = jnp.minimum(
            mna_ref[...],
            jnp.min(jnp.where(oh > 0, imp2, _BIG), axis=0, keepdims=True))
        mxa_ref[...] = jnp.maximum(
            mxa_ref[...],
            jnp.max(jnp.where(oh > 0, imp2, -_BIG), axis=0, keepdims=True))
    return pl.pallas_call(
        body,
        grid=(_NB,),
        in_specs=[
            pl.BlockSpec((_BN, _HH), lambda i: (i, 0)),
            pl.BlockSpec((_BN, _HH), lambda i: (_NB + i, 0)),
            pl.BlockSpec((_BN, _HH), lambda i: (i, 0)),
            pl.BlockSpec((_BN, _HH), lambda i: (_NB + i, 0)),
            pl.BlockSpec((_BN, 1), lambda i: (i, 0)),
            pl.BlockSpec((1, _B), lambda i: (0, 0)),
            pl.BlockSpec((1, _B), lambda i: (0, 0)),
            pl.BlockSpec((_BN, _B), lambda i: (i, 0)),
            pl.BlockSpec((_H, _H), lambda i: (0, 0)),
            pl.BlockSpec((1, _H), lambda i: (0, 0)),
            pl.BlockSpec((_H, _H), lambda i: (0, 0)),
            pl.BlockSpec((1, _H), lambda i: (0, 0)),
            pl.BlockSpec((1, _H), lambda i: (0, 0)),
            pl.BlockSpec((1, _H), lambda i: (0, 0)),
            pl.BlockSpec((1, 1), lambda i: (0, 0)),
        ],
        out_specs=[
            pl.BlockSpec((_BN, _H), lambda i: (i, 0)),
            pl.BlockSpec((_BN, 1), lambda i: (i, 0)),
            pl.BlockSpec((_B, _H), lambda i: (0, 0)),
            pl.BlockSpec((_B, _H), lambda i: (0, 0)),
            pl.BlockSpec((1, _B), lambda i: (0, 0)),
            pl.BlockSpec((1, _B), lambda i: (0, 0)),
        ],
        out_shape=[
            jax.ShapeDtypeStruct((_N, _H), jnp.float32),
            jax.ShapeDtypeStruct((_N, 1), jnp.float32),
            jax.ShapeDtypeStruct((_B, _H), jnp.float32),
            jax.ShapeDtypeStruct((_B, _H), jnp.float32),
            jax.ShapeDtypeStruct((1, _B), jnp.float32),
            jax.ShapeDtypeStruct((1, _B), jnp.float32),
        ],
    )(hs, hs, aggs, aggs, imp_raw, mn, mx, oh,
      convW, convb, gWh, gWi, gb, propWr, propb)


def _layer_k23(out, seg, segq, cnt, oh, gn_ms, gn_w, gn_b, imp2, mna, mxa):
    def body(out_ref, seg_ref, segq_ref, cnt_ref, oh_ref,
             ms_ref, gw_ref, gb_ref, imp2_ref, mna_ref, mxa_ref,
             hs_ref, imp3_ref, mnb_ref, mxb_ref):
        c = pl.program_id(0)
        i = pl.program_id(1)
        ms = ms_ref[...]
        mean = seg_ref[...] / cnt_ref[...]
        var = segq_ref[...] / cnt_ref[...] - mean * mean * ms * (2.0 - ms)
        oh = oh_ref[...]
        meanb = jnp.dot(oh, mean, preferred_element_type=jnp.float32)
        varb = jnp.dot(oh, var, preferred_element_type=jnp.float32)
        cent = out_ref[...] - meanb * ms
        normed = cent * lax.rsqrt(varb + 1e-5) * gw_ref[...] + gb_ref[...]
        hs_ref[...] = jnp.maximum(normed, 0.0)
        imp2n = _lazy_norm(imp2_ref[...], mna_ref[...], mxa_ref[...], oh)
        imp3 = jnp.where(imp2n < _THRESH, 0.0, imp2n)
        imp3_ref[...] = imp3
        @pl.when((c == 0) & (i == 0))
        def _():
            mnb_ref[...] = jnp.full((1, _B), _BIG, jnp.float32)
            mxb_ref[...] = jnp.full((1, _B), -_BIG, jnp.float32)
        mnb_ref[...] = jnp.minimum(
            mnb_ref[...],
            jnp.min(jnp.where(oh > 0, imp3, _BIG), axis=0, keepdims=True))
        mxb_ref[...] = jnp.maximum(
            mxb_ref[...],
            jnp.max(jnp.where(oh > 0, imp3, -_BIG), axis=0, keepdims=True))
    return pl.pallas_call(
        body,
        grid=(2, _NB),
        in_specs=[
            pl.BlockSpec((_BN, _HH), lambda c, i: (i, c)),
            pl.BlockSpec((_B, _HH), lambda c, i: (0, c)),
            pl.BlockSpec((_B, _HH), lambda c, i: (0, c)),
            pl.BlockSpec((_B, 1), lambda c, i: (0, 0)),
            pl.BlockSpec((_BN, _B), lambda c, i: (i, 0)),
            pl.BlockSpec((1, _HH), lambda c, i: (0, c)),
            pl.BlockSpec((1, _HH), lambda c, i: (0, c)),
            pl.BlockSpec((1, _HH), lambda c, i: (0, c)),
            pl.BlockSpec((_BN, 1), lambda c, i: (i, 0)),
            pl.BlockSpec((1, _B), lambda c, i: (0, 0)),
            pl.BlockSpec((1, _B), lambda c, i: (0, 0)),
        ],
        out_specs=[
            pl.BlockSpec((_BN, _HH), lambda c, i: (c * _NB + i, 0)),
            pl.BlockSpec((_BN, 1), lambda c, i: (i, 0)),
            pl.BlockSpec((1, _B), lambda c, i: (0, 0)),
            pl.BlockSpec((1, _B), lambda c, i: (0, 0)),
        ],
        out_shape=[
            jax.ShapeDtypeStruct((2 * _N, _HH), jnp.float32),
            jax.ShapeDtypeStruct((_N, 1), jnp.float32),
            jax.ShapeDtypeStruct((1, _B), jnp.float32),
            jax.ShapeDtypeStruct((1, _B), jnp.float32),
        ],
    )(out, seg, segq, cnt, oh, gn_ms, gn_w, gn_b, imp2, mna, mxa)


def _final_k1(hs, finalW, finalb, oh, imp0, imp3, mn, mx):
    def body(hlo, hhi, w_ref, b_ref, oh_ref, imp0_ref,
             imp3_ref, mn_ref, mx_ref, gsum_ref, s_ref, mnf_ref, mxf_ref):
        i = pl.program_id(0)
        h = jnp.concatenate([hlo[...], hhi[...]], axis=1)
        hf = jnp.dot(h, w_ref[...], preferred_element_type=jnp.float32) + b_ref[...]
        imp = _lazy_norm(imp3_ref[...], mn_ref[...], mx_ref[...], oh_ref[...])
        s = imp0_ref[...] + imp
        s_ref[...] = s
        oh = oh_ref[...]
        @pl.when(i == 0)
        def _():
            gsum_ref[...] = jnp.zeros((_B, _OUT), jnp.float32)
            mnf_ref[...] = jnp.full((1, _B), _BIG, jnp.float32)
            mxf_ref[...] = jnp.full((1, _B), -_BIG, jnp.float32)
        gsum_ref[...] += _segdot(oh, hf)
        mnf_ref[...] = jnp.minimum(
            mnf_ref[...],
            jnp.min(jnp.where(oh > 0, s, _BIG), axis=0, keepdims=True))
        mxf_ref[...] = jnp.maximum(
            mxf_ref[...],
            jnp.max(jnp.where(oh > 0, s, -_BIG), axis=0, keepdims=True))
    return pl.pallas_call(
        body,
        grid=(_NB,),
        in_specs=[
            pl.BlockSpec((_BN, _HH), lambda i: (i, 0)),
            pl.BlockSpec((_BN, _HH), lambda i: (_NB + i, 0)),
            pl.BlockSpec((_H, _OUT), lambda i: (0, 0)),
            pl.BlockSpec((1, _OUT), lambda i: (0, 0)),
            pl.BlockSpec((_BN, _B), lambda i: (i, 0)),
            pl.BlockSpec((_BN, 1), lambda i: (i, 0)),
            pl.BlockSpec((_BN, 1), lambda i: (i, 0)),
            pl.BlockSpec((1, _B), lambda i: (0, 0)),
            pl.BlockSpec((1, _B), lambda i: (0, 0)),
        ],
        out_specs=[
            pl.BlockSpec((_B, _OUT), lambda i: (0, 0)),
            pl.BlockSpec((_BN, 1), lambda i: (i, 0)),
            pl.BlockSpec((1, _B), lambda i: (0, 0)),
            pl.BlockSpec((1, _B), lambda i: (0, 0)),
        ],
        out_shape=[
            jax.ShapeDtypeStruct((_B, _OUT), jnp.float32),
            jax.ShapeDtypeStruct((_N, 1), jnp.float32),
            jax.ShapeDtypeStruct((1, _B), jnp.float32),
            jax.ShapeDtypeStruct((1, _B), jnp.float32),
        ],
    )(hs, hs, finalW, finalb, oh, imp0, imp3, mn, mx)


def _final_k2(gsum, cnt, s, oh, mnf, mxf):
    def body(gsum_ref, cnt_ref, s_ref, oh_ref, mn_ref, mx_ref,
             gemb_ref, fi_ref):
        gemb_ref[...] = gsum_ref[...] / cnt_ref[...]
        fi_ref[...] = _lazy_norm(s_ref[...], mn_ref[...], mx_ref[...],
                                 oh_ref[...])
    return pl.pallas_call(
        body,
        grid=(_NB,),
        in_specs=[
            pl.BlockSpec((_B, _OUT), lambda i: (0, 0)),
            pl.BlockSpec((_B, 1), lambda i: (0, 0)),
            pl.BlockSpec((_BN, 1), lambda i: (i, 0)),
            pl.BlockSpec((_BN, _B), lambda i: (i, 0)),
            pl.BlockSpec((1, _B), lambda i: (0, 0)),
            pl.BlockSpec((1, _B), lambda i: (0, 0)),
        ],
        out_specs=[
            pl.BlockSpec((_B, _OUT), lambda i: (0, 0)),
            pl.BlockSpec((_BN, 1), lambda i: (i, 0)),
        ],
        out_shape=[
            jax.ShapeDtypeStruct((_B, _OUT), jnp.float32),
            jax.ShapeDtypeStruct((_N, 1), jnp.float32),
        ],
    )(gsum, cnt, s, oh, mnf, mxf)


def kernel(x, edge_index, edge_attr, initial_importance, batch,
           imp_proj_W, imp_proj_b, init_W, init_b, conv_lin_W, conv_lin_b,
           edge_lin_W, edge_lin_b, gate_W, gate_b, prop_W, prop_b,
           gn_weight, gn_bias, gn_mean_scale, final_W, final_b):
    src = edge_index[0]
    dst = edge_index[1]
    imp0 = initial_importance[:, None]
    oh = (batch[:, None] == jnp.arange(_B, dtype=batch.dtype)[None, :]
          ).astype(jnp.float32)

    We = edge_lin_W.reshape(_L, _ED, 2, _HH).transpose(0, 2, 1, 3)
    be = edge_lin_b.reshape(_L, 2, 1, _HH)

    mn, mx, cnt_row = _seg_stats0(imp0, oh)
    cnt = cnt_row.reshape(_B, 1)
    hs = _init_h(x, imp0, mn, mx, oh,
                 imp_proj_W.reshape(1, _H), imp_proj_b.reshape(1, _H),
                 init_W[:_D], init_W[_D:], init_b.reshape(1, _H))
    imp_raw = imp0

    for l in range(_L):
        e_l = _edge_proj(edge_attr, We[l], be[l]).reshape(2 * _E, _HH)
        aggs = _sc_agg(hs, e_l, src, dst)
        out, imp2, seg, segq, mna, mxa = _layer_k1(
            hs, aggs, imp_raw, mn, mx, oh,
            conv_lin_W[l], conv_lin_b[l].reshape(1, _H),
            gate_W[l][:_H], gate_W[l][_H:].reshape(1, _H),
            gate_b[l].reshape(1, _H),
            prop_W[l].reshape(1, _H), prop_b[l].reshape(1, 1))
        hs, imp_raw, mn, mx = _layer_k23(
            out, seg, segq, cnt, oh, gn_mean_scale[l].reshape(1, _H),
            gn_weight[l].reshape(1, _H), gn_bias[l].reshape(1, _H),
            imp2, mna, mxa)

    gsum, s, mnf, mxf = _final_k1(hs, final_W, final_b.reshape(1, _OUT),
                                  oh, imp0, imp_raw, mn, mx)
    graph_emb, final_imp = _final_k2(gsum, cnt, s, oh, mnf, mxf)
    return (graph_emb, final_imp)
